# baseline (speedup 1.0000x reference)
import numpy as np

import jax
import jax.numpy as jnp
from jax import lax
from jax.experimental import pallas as pl
from jax.experimental.pallas import tpu as pltpu
from jax.experimental.pallas import tpu_sc as plsc

_B = 16384
_F = 26
_D = 32
_V = 1000000
_L = 16
_NW = 32
_RPW = _B // _NW
_CHUNK = 16
_NCHUNK = _RPW // _CHUNK
_IPC = _CHUNK * _F
_SROW = 128
_NSUP = _V * _D // _SROW
_NBLK = _V // _SROW
_W = 512
_NSB = _V // _W - 1
_KPT = _NSB // _NW

def _tbody(embt_hbm, tail_hbm, sup_hbm, in0_v, in1_v, out0_v, out1_v,
           sem_i0, sem_i1, sem_o0, sem_o1):
    nc = 2
    wid = lax.axis_index("s") * nc + lax.axis_index("c")
    iota = lax.iota(jnp.int32, _L)
    basek, orowc, ocolc = [], [], []
    for k in range(16):
        cv = (iota + k) & 15
        basek.append(cv)
        orow_k, ocol_k = [], []
        for dh in range(2):
            flat = cv * _D + (iota + dh * 16)
            orow_k.append(lax.shift_right_logical(flat, 7))
            ocol_k.append(flat & 127)
        orowc.append(orow_k)
        ocolc.append(ocol_k)
    sems_i = (sem_i0, sem_i1)
    sems_o = (sem_o0, sem_o1)
    ins = (in0_v, in1_v)
    outs = (out0_v, out1_v)

    def issue_in(slot, sb, w=_W):
        for dg in range(4):
            pltpu.async_copy(
                embt_hbm.at[pl.ds(dg * 8, 8), pl.ds(sb * _W, w)],
                ins[slot].at[pl.ds(dg * 8, 8), pl.ds(0, w)], sems_i[slot])

    def wait_in(slot, w=_W):
        for dg in range(4):
            pltpu.make_async_copy(
                embt_hbm.at[pl.ds(0, 8), pl.ds(0, w)],
                ins[slot].at[pl.ds(dg * 8, 8), pl.ds(0, w)],
                sems_i[slot]).wait()

    def compute(slot, nsb=_W // _L):
        @pl.loop(0, nsb)
        def _isb(isb):
            isb16 = isb * 16
            isb4 = isb * 4
            for dh in range(2):
                rv = iota + dh * 16
                for k in range(16):
                    cvec = basek[k] + isb16
                    orow = orowc[k][dh] + isb4
                    v = plsc.load_gather(ins[slot], [rv, cvec])
                    plsc.store_scatter(
                        outs[slot], [orow, ocolc[k][dh]], v)

    def issue_out(slot, srow0, nrow=_W // 4):
        pltpu.async_copy(outs[slot].at[pl.ds(0, nrow), :],
                         sup_hbm.at[pl.ds(srow0, nrow), :], sems_o[slot])

    def wait_out(slot, nrow=_W // 4):
        pltpu.make_async_copy(outs[slot].at[pl.ds(0, nrow), :],
                              sup_hbm.at[pl.ds(0, nrow), :],
                              sems_o[slot]).wait()

    issue_in(0, wid)
    issue_in(1, wid + _NW)

    @pl.loop(0, (_KPT + 1) // 2)
    def _pair(kk):
        for s in range(2):
            k = kk * 2 + s

            @pl.when(k < _KPT)
            def _():
                sb = wid + k * _NW
                wait_in(s)

                @pl.when(k >= 2)
                def _():
                    wait_out(s)

                compute(s)
                issue_out(s, sb * (_W // 4))

                @pl.when(k + 2 < _KPT)
                def _():
                    issue_in(s, wid + (k + 2) * _NW)

    wait_out(0)
    wait_out(1)

    @pl.when(wid < 4)
    def _extra():
        i0 = _NSB * _W // _SROW + wid
        for dg in range(4):
            pltpu.async_copy(
                embt_hbm.at[pl.ds(dg * 8, 8), pl.ds(i0 * _SROW, _SROW)],
                in0_v.at[pl.ds(dg * 8, 8), pl.ds(0, _SROW)], sem_i0)
        for dg in range(4):
            pltpu.make_async_copy(
                embt_hbm.at[pl.ds(0, 8), pl.ds(0, _SROW)],
                in0_v.at[pl.ds(dg * 8, 8), pl.ds(0, _SROW)],
                sem_i0).wait()
        compute(0, nsb=_SROW // _L)
        issue_out(0, i0 * 32, nrow=32)
        wait_out(0, nrow=32)

    @pl.when(wid == 4)
    def _tail():
        pltpu.sync_copy(tail_hbm, in1_v.at[pl.ds(0, 16), pl.ds(0, _SROW)])
        pltpu.sync_copy(in1_v.at[pl.ds(0, 16), pl.ds(0, _SROW)],
                        sup_hbm.at[pl.ds(_NSUP - 16, 16), :])


def _gbody(feat_hbm, fv_hbm, sup_hbm, bias_hbm, out_hbm,
           idx_v, idxq_v, emb_v, bias_v, fv_v, out_v, sem_e, sem_b):
    nc = 2
    wid = lax.axis_index("s") * nc + lax.axis_index("c")
    iota = lax.iota(jnp.int32, _L)
    m1 = iota < (_F - _L)
    mlast = iota == (_L - 1)
    zero_v = jnp.zeros((_L,), jnp.float32)

    @pl.loop(0, _NCHUNK)
    def _chunk(c):
        i0 = wid * (_NCHUNK * _IPC) + c * _IPC
        pltpu.sync_copy(feat_hbm.at[pl.ds(i0, _IPC)], idx_v)
        pltpu.sync_copy(fv_hbm.at[pl.ds(i0, _IPC)], fv_v)
        for j in range(_IPC // _L):
            idxq_v[pl.ds(j * _L, _L)] = (
                lax.shift_right_logical(idx_v[pl.ds(j * _L, _L)], 2))
        descs = []
        for j0 in range(0, _IPC, 128):
            n = min(128, _IPC - j0)
            descs.append(pltpu.async_copy(
                sup_hbm.at[idxq_v.at[pl.ds(j0, n)]],
                emb_v.at[pl.ds(j0, n), :], sem_e))
            descs.append(pltpu.async_copy(
                bias_hbm.at[idx_v.at[pl.ds(j0, n)]],
                bias_v.at[pl.ds(j0, n)], sem_b))
        for d in descs:
            d.wait()

        @pl.loop(0, _CHUNK)
        def _row(b):
            r0 = b * _F
            ix0 = r0 + iota
            ix1 = ix0 + _L
            vv0 = plsc.load_gather(fv_v, [ix0])
            vv1 = plsc.load_gather(fv_v, [ix1], mask=m1)
            qo0 = (plsc.load_gather(idx_v, [ix0]) & 3) * _D
            qo1 = (plsc.load_gather(idx_v, [ix1], mask=m1) & 3) * _D
            acc0 = zero_v
            acc1 = zero_v
            sq0 = zero_v
            sq1 = zero_v
            rsplat = jnp.full((_L,), 0, jnp.int32) + r0
            for f in range(_F):
                sv = (vv0 if f < _L else vv1)[f % _L]
                co = (qo0 if f < _L else qo1)[f % _L]
                rvec = rsplat + f
                c0 = co + iota
                e0 = plsc.load_gather(emb_v, [rvec, c0])
                e1 = plsc.load_gather(emb_v, [rvec, c0 + _L])
                ev0 = e0 * sv
                ev1 = e1 * sv
                acc0 = acc0 + ev0
                acc1 = acc1 + ev1
                sq0 = sq0 + ev0 * ev0
                sq1 = sq1 + ev1 * ev1
            fm = acc0 * acc0 - sq0 + acc1 * acc1 - sq1
            bb0 = plsc.load_gather(bias_v, [ix0])
            bb1 = plsc.load_gather(bias_v, [ix1], mask=m1)
            bvec = bb0 * vv0 + jnp.where(m1, bb1 * vv1, 0.0)
            t = lax.reduce_sum(0.5 * fm + bvec, axes=(0,))
            tv = jnp.full((_L,), 0.0, jnp.float32) + t
            plsc.store_scatter(out_v, [jnp.full((_L,), 0, jnp.int32) + b],
                               tv, mask=mlast)

        pltpu.sync_copy(out_v, out_hbm.at[pl.ds(wid * _RPW + c * _CHUNK,
                                                _CHUNK)])


_MESH = plsc.VectorSubcoreMesh(core_axis_name="c", subcore_axis_name="s")
_PARAMS = pltpu.CompilerParams(
    needs_layout_passes=False, use_tc_tiling_on_sc=True)


@jax.jit
def _pointfm_sc(feat_flat, fv_flat, emb_t, tail16, bias_flat):
    sup = pl.kernel(
        _tbody,
        out_type=jax.ShapeDtypeStruct((_NSUP, _SROW), jnp.float32),
        mesh=_MESH,
        scratch_types=[
            pltpu.VMEM((_D, _W), jnp.float32),
            pltpu.VMEM((_D, _W), jnp.float32),
            pltpu.VMEM((_W // 4, _SROW), jnp.float32),
            pltpu.VMEM((_W // 4, _SROW), jnp.float32),
            pltpu.SemaphoreType.DMA,
            pltpu.SemaphoreType.DMA,
            pltpu.SemaphoreType.DMA,
            pltpu.SemaphoreType.DMA,
        ],
        compiler_params=_PARAMS,
    )(emb_t, tail16)
    return pl.kernel(
        _gbody,
        out_type=jax.ShapeDtypeStruct((_B,), jnp.float32),
        mesh=_MESH,
        scratch_types=[
            pltpu.VMEM((_IPC,), jnp.int32),
            pltpu.VMEM((_IPC,), jnp.int32),
            pltpu.VMEM((_IPC, _SROW), jnp.float32),
            pltpu.VMEM((_IPC,), jnp.float32),
            pltpu.VMEM((_IPC,), jnp.float32),
            pltpu.VMEM((_CHUNK,), jnp.float32),
            pltpu.SemaphoreType.DMA,
            pltpu.SemaphoreType.DMA,
        ],
        compiler_params=_PARAMS,
    )(feat_flat, fv_flat, sup, bias_flat)


def kernel(features, feature_values, emb_table, bias_table, bias_):
    tail16 = emb_table[_V - 2 * _D:].reshape(16, _SROW)
    out = _pointfm_sc(features.reshape(-1), feature_values.reshape(-1),
                      emb_table.T, tail16, bias_table.reshape(-1))
    return out + bias_


# speedup vs baseline: 1.1889x; 1.1889x over previous
import numpy as np

import jax
import jax.numpy as jnp
from jax import lax
from jax.experimental import pallas as pl
from jax.experimental.pallas import tpu as pltpu
from jax.experimental.pallas import tpu_sc as plsc

_B = 16384
_F = 26
_D = 32
_V = 1000000
_L = 16
_NW = 32
_RPW = _B // _NW
_CHUNK = 16
_NCHUNK = _RPW // _CHUNK
_IPC = _CHUNK * _F
_SROW = 128
_NSUP = _V * _D // _SROW
_NBLK = _V // _SROW
_W = 512
_NSB = _V // _W - 1
_KPT = _NSB // _NW

def _tbody(embt_hbm, tail_hbm, sup_hbm, in0_v, in1_v, out0_v, out1_v,
           sem_i0, sem_i1, sem_o0, sem_o1):
    nc = 2
    wid = lax.axis_index("s") * nc + lax.axis_index("c")
    iota = lax.iota(jnp.int32, _L)
    basek, orowc, ocolc = [], [], []
    for k in range(16):
        cv = (iota + k) & 15
        basek.append(cv)
        orow_k, ocol_k = [], []
        for dh in range(2):
            flat = cv * _D + (iota + dh * 16)
            orow_k.append(lax.shift_right_logical(flat, 7))
            ocol_k.append(flat & 127)
        orowc.append(orow_k)
        ocolc.append(ocol_k)
    sems_i = (sem_i0, sem_i1)
    sems_o = (sem_o0, sem_o1)
    ins = (in0_v, in1_v)
    outs = (out0_v, out1_v)

    def issue_in(slot, sb, w=_W):
        for dg in range(4):
            pltpu.async_copy(
                embt_hbm.at[pl.ds(dg * 8, 8), pl.ds(sb * _W, w)],
                ins[slot].at[pl.ds(dg * 8, 8), pl.ds(0, w)], sems_i[slot])

    def wait_in(slot, w=_W):
        for dg in range(4):
            pltpu.make_async_copy(
                embt_hbm.at[pl.ds(0, 8), pl.ds(0, w)],
                ins[slot].at[pl.ds(dg * 8, 8), pl.ds(0, w)],
                sems_i[slot]).wait()

    def compute(slot, nsb=_W // _L):
        @pl.loop(0, nsb)
        def _isb(isb):
            isb16 = isb * 16
            isb4 = isb * 4
            for dh in range(2):
                rv = iota + dh * 16
                for k in range(16):
                    cvec = basek[k] + isb16
                    orow = orowc[k][dh] + isb4
                    v = plsc.load_gather(ins[slot], [rv, cvec])
                    plsc.store_scatter(
                        outs[slot], [orow, ocolc[k][dh]], v)

    def issue_out(slot, srow0, nrow=_W // 4):
        pltpu.async_copy(outs[slot].at[pl.ds(0, nrow), :],
                         sup_hbm.at[pl.ds(srow0, nrow), :], sems_o[slot])

    def wait_out(slot, nrow=_W // 4):
        pltpu.make_async_copy(outs[slot].at[pl.ds(0, nrow), :],
                              sup_hbm.at[pl.ds(0, nrow), :],
                              sems_o[slot]).wait()

    issue_in(0, wid)
    issue_in(1, wid + _NW)

    @pl.loop(0, (_KPT + 1) // 2)
    def _pair(kk):
        for s in range(2):
            k = kk * 2 + s

            @pl.when(k < _KPT)
            def _():
                sb = wid + k * _NW
                wait_in(s)

                @pl.when(k >= 2)
                def _():
                    wait_out(s)

                compute(s)
                issue_out(s, sb * (_W // 4))

                @pl.when(k + 2 < _KPT)
                def _():
                    issue_in(s, wid + (k + 2) * _NW)

    wait_out(0)
    wait_out(1)

    @pl.when(wid < 4)
    def _extra():
        i0 = _NSB * _W // _SROW + wid
        for dg in range(4):
            pltpu.async_copy(
                embt_hbm.at[pl.ds(dg * 8, 8), pl.ds(i0 * _SROW, _SROW)],
                in0_v.at[pl.ds(dg * 8, 8), pl.ds(0, _SROW)], sem_i0)
        for dg in range(4):
            pltpu.make_async_copy(
                embt_hbm.at[pl.ds(0, 8), pl.ds(0, _SROW)],
                in0_v.at[pl.ds(dg * 8, 8), pl.ds(0, _SROW)],
                sem_i0).wait()
        compute(0, nsb=_SROW // _L)
        issue_out(0, i0 * 32, nrow=32)
        wait_out(0, nrow=32)

    @pl.when(wid == 4)
    def _tail():
        pltpu.sync_copy(tail_hbm, in1_v.at[pl.ds(0, 16), pl.ds(0, _SROW)])
        pltpu.sync_copy(in1_v.at[pl.ds(0, 16), pl.ds(0, _SROW)],
                        sup_hbm.at[pl.ds(_NSUP - 16, 16), :])


def _gbody(feat_hbm, fv_hbm, sup_hbm, bias_hbm, out_hbm,
           idx0_v, idx1_v, idxq0_v, idxq1_v, emb0_v, emb1_v,
           bias0_v, bias1_v, fv0_v, fv1_v, out_v,
           sem_e0, sem_e1, sem_b0, sem_b1):
    nc = 2
    wid = lax.axis_index("s") * nc + lax.axis_index("c")
    iota = lax.iota(jnp.int32, _L)
    m1 = iota < (_F - _L)
    mlast = iota == (_L - 1)
    zero_v = jnp.zeros((_L,), jnp.float32)
    sems_e = (sem_e0, sem_e1)
    sems_b = (sem_b0, sem_b1)
    idxs = (idx0_v, idx1_v)
    idxqs = (idxq0_v, idxq1_v)
    embs = (emb0_v, emb1_v)
    biass = (bias0_v, bias1_v)
    fvs = (fv0_v, fv1_v)

    def stage(s, c):
        i0 = wid * (_NCHUNK * _IPC) + c * _IPC
        pltpu.sync_copy(feat_hbm.at[pl.ds(i0, _IPC)], idxs[s])
        pltpu.sync_copy(fv_hbm.at[pl.ds(i0, _IPC)], fvs[s])
        for j in range(_IPC // _L):
            idxqs[s][pl.ds(j * _L, _L)] = (
                lax.shift_right_logical(idxs[s][pl.ds(j * _L, _L)], 2))
        for j0 in range(0, _IPC, 128):
            n = min(128, _IPC - j0)
            pltpu.async_copy(
                sup_hbm.at[idxqs[s].at[pl.ds(j0, n)]],
                embs[s].at[pl.ds(j0, n), :], sems_e[s])
            pltpu.async_copy(
                bias_hbm.at[idxs[s].at[pl.ds(j0, n)]],
                biass[s].at[pl.ds(j0, n)], sems_b[s])

    def drain(s):
        for j0 in range(0, _IPC, 128):
            n = min(128, _IPC - j0)
            pltpu.make_async_copy(
                sup_hbm.at[pl.ds(0, n), :],
                embs[s].at[pl.ds(j0, n), :], sems_e[s]).wait()
            pltpu.make_async_copy(
                bias_hbm.at[pl.ds(0, n)],
                biass[s].at[pl.ds(j0, n)], sems_b[s]).wait()

    stage(0, 0)

    @pl.loop(0, _NCHUNK // 2)
    def _chunkpair(cc):
      for s in range(2):
        c = cc * 2 + s

        @pl.when(c + 1 < _NCHUNK)
        def _():
            stage(1 - s, c + 1)

        drain(s)
        idx_c = idxs[s]
        fv_c = fvs[s]
        emb_c = embs[s]
        bias_c = biass[s]

        @pl.loop(0, _CHUNK)
        def _row(b):
            r0 = b * _F
            ix0 = r0 + iota
            ix1 = ix0 + _L
            vv0 = plsc.load_gather(fv_c, [ix0])
            vv1 = plsc.load_gather(fv_c, [ix1], mask=m1)
            qo0 = (plsc.load_gather(idx_c, [ix0]) & 3) * _D
            qo1 = (plsc.load_gather(idx_c, [ix1], mask=m1) & 3) * _D
            acc0 = zero_v
            acc1 = zero_v
            sq0 = zero_v
            sq1 = zero_v
            rsplat = jnp.full((_L,), 0, jnp.int32) + r0
            for f in range(_F):
                sv = (vv0 if f < _L else vv1)[f % _L]
                co = (qo0 if f < _L else qo1)[f % _L]
                rvec = rsplat + f
                c0 = co + iota
                e0 = plsc.load_gather(emb_c, [rvec, c0])
                e1 = plsc.load_gather(emb_c, [rvec, c0 + _L])
                ev0 = e0 * sv
                ev1 = e1 * sv
                acc0 = acc0 + ev0
                acc1 = acc1 + ev1
                sq0 = sq0 + ev0 * ev0
                sq1 = sq1 + ev1 * ev1
            fm = acc0 * acc0 - sq0 + acc1 * acc1 - sq1
            bb0 = plsc.load_gather(bias_c, [ix0])
            bb1 = plsc.load_gather(bias_c, [ix1], mask=m1)
            bvec = bb0 * vv0 + jnp.where(m1, bb1 * vv1, 0.0)
            t = lax.reduce_sum(0.5 * fm + bvec, axes=(0,))
            tv = jnp.full((_L,), 0.0, jnp.float32) + t
            plsc.store_scatter(out_v, [jnp.full((_L,), 0, jnp.int32) + b],
                               tv, mask=mlast)

        pltpu.sync_copy(out_v, out_hbm.at[pl.ds(wid * _RPW + c * _CHUNK,
                                                _CHUNK)])


_MESH = plsc.VectorSubcoreMesh(core_axis_name="c", subcore_axis_name="s")
_PARAMS = pltpu.CompilerParams(
    needs_layout_passes=False, use_tc_tiling_on_sc=True)


@jax.jit
def _pointfm_sc(feat_flat, fv_flat, emb_t, tail16, bias_flat):
    sup = pl.kernel(
        _tbody,
        out_type=jax.ShapeDtypeStruct((_NSUP, _SROW), jnp.float32),
        mesh=_MESH,
        scratch_types=[
            pltpu.VMEM((_D, _W), jnp.float32),
            pltpu.VMEM((_D, _W), jnp.float32),
            pltpu.VMEM((_W // 4, _SROW), jnp.float32),
            pltpu.VMEM((_W // 4, _SROW), jnp.float32),
            pltpu.SemaphoreType.DMA,
            pltpu.SemaphoreType.DMA,
            pltpu.SemaphoreType.DMA,
            pltpu.SemaphoreType.DMA,
        ],
        compiler_params=_PARAMS,
    )(emb_t, tail16)
    return pl.kernel(
        _gbody,
        out_type=jax.ShapeDtypeStruct((_B,), jnp.float32),
        mesh=_MESH,
        scratch_types=[
            pltpu.VMEM((_IPC,), jnp.int32),
            pltpu.VMEM((_IPC,), jnp.int32),
            pltpu.VMEM((_IPC,), jnp.int32),
            pltpu.VMEM((_IPC,), jnp.int32),
            pltpu.VMEM((_IPC, _SROW), jnp.float32),
            pltpu.VMEM((_IPC, _SROW), jnp.float32),
            pltpu.VMEM((_IPC,), jnp.float32),
            pltpu.VMEM((_IPC,), jnp.float32),
            pltpu.VMEM((_IPC,), jnp.float32),
            pltpu.VMEM((_IPC,), jnp.float32),
            pltpu.VMEM((_CHUNK,), jnp.float32),
            pltpu.SemaphoreType.DMA,
            pltpu.SemaphoreType.DMA,
            pltpu.SemaphoreType.DMA,
            pltpu.SemaphoreType.DMA,
        ],
        compiler_params=_PARAMS,
    )(feat_flat, fv_flat, sup, bias_flat)


def kernel(features, feature_values, emb_table, bias_table, bias_):
    tail16 = emb_table[_V - 2 * _D:].reshape(16, _SROW)
    out = _pointfm_sc(features.reshape(-1), feature_values.reshape(-1),
                      emb_table.T, tail16, bias_table.reshape(-1))
    return out + bias_


# speedup vs baseline: 1.2099x; 1.0177x over previous
import jax
import jax.numpy as jnp
from jax import lax
from jax.experimental import pallas as pl
from jax.experimental.pallas import tpu as pltpu
from jax.experimental.pallas import tpu_sc as plsc

_B = 16384
_F = 26
_D = 32
_V = 1000000
_L = 16
_NW = 32
_RPW = _B // _NW
_CHUNK = 16
_NCHUNK = _RPW // _CHUNK
_IPC = _CHUNK * _F
_SROW = 128
_NSUP = _V * _D // _SROW
_NBLK = _V // _SROW
_W = 256
_NSB = (_V // _W // _NW) * _NW
_KPT = _NSB // _NW
_NEXTRA = (_V - 64) // _W - _NSB
_NSLOT = 4

def _tbody(embt_hbm, tail_hbm, sup_hbm, in0_v, in1_v, in2_v, in3_v,
           out0_v, out1_v, out2_v, out3_v,
           sem_i0, sem_i1, sem_i2, sem_i3,
           sem_o0, sem_o1, sem_o2, sem_o3):
    nc = 2
    wid = lax.axis_index("s") * nc + lax.axis_index("c")
    iota = lax.iota(jnp.int32, _L)
    basek, orowc, ocolc = [], [], []
    for k in range(16):
        cv = (iota + k) & 15
        basek.append(cv)
        orow_k, ocol_k = [], []
        for dh in range(2):
            flat = cv * _D + (iota + dh * 16)
            orow_k.append(lax.shift_right_logical(flat, 7))
            ocol_k.append(flat & 127)
        orowc.append(orow_k)
        ocolc.append(ocol_k)
    sems_i = (sem_i0, sem_i1, sem_i2, sem_i3)
    sems_o = (sem_o0, sem_o1, sem_o2, sem_o3)
    ins = (in0_v, in1_v, in2_v, in3_v)
    outs = (out0_v, out1_v, out2_v, out3_v)

    def issue_in(slot, sb, w=_W):
        for dg in range(4):
            pltpu.async_copy(
                embt_hbm.at[pl.ds(dg * 8, 8), pl.ds(sb * _W, w)],
                ins[slot].at[pl.ds(dg * 8, 8), pl.ds(0, w)], sems_i[slot])

    def wait_in(slot, w=_W):
        for dg in range(4):
            pltpu.make_async_copy(
                embt_hbm.at[pl.ds(0, 8), pl.ds(0, w)],
                ins[slot].at[pl.ds(dg * 8, 8), pl.ds(0, w)],
                sems_i[slot]).wait()

    def compute(slot, nsb=_W // _L):
        @pl.loop(0, nsb)
        def _isb(isb):
            isb16 = isb * 16
            isb4 = isb * 4
            for dh in range(2):
                rv = iota + dh * 16
                for k in range(16):
                    cvec = basek[k] + isb16
                    orow = orowc[k][dh] + isb4
                    v = plsc.load_gather(ins[slot], [rv, cvec])
                    plsc.store_scatter(
                        outs[slot], [orow, ocolc[k][dh]], v)

    def issue_out(slot, srow0, nrow=_W // 4):
        pltpu.async_copy(outs[slot].at[pl.ds(0, nrow), :],
                         sup_hbm.at[pl.ds(srow0, nrow), :], sems_o[slot])

    def wait_out(slot, nrow=_W // 4):
        pltpu.make_async_copy(outs[slot].at[pl.ds(0, nrow), :],
                              sup_hbm.at[pl.ds(0, nrow), :],
                              sems_o[slot]).wait()

    for s in range(_NSLOT):
        issue_in(s, wid + s * _NW)

    @pl.loop(0, (_KPT + _NSLOT - 1) // _NSLOT)
    def _round(kk):
        for s in range(_NSLOT):
            k = kk * _NSLOT + s

            @pl.when(k < _KPT)
            def _():
                sb = wid + k * _NW
                wait_in(s)

                @pl.when(k >= _NSLOT)
                def _():
                    wait_out(s)

                compute(s)
                issue_out(s, sb * (_W // 4))

                @pl.when(k + _NSLOT < _KPT)
                def _():
                    issue_in(s, wid + (k + _NSLOT) * _NW)

    for s in range(_NSLOT):
        wait_out(s)

    @pl.when(wid < _NEXTRA)
    def _extra():
        sb = _NSB + wid
        issue_in(0, sb)
        wait_in(0)
        compute(0)
        issue_out(0, sb * (_W // 4))
        wait_out(0)

    @pl.when(wid == 4)
    def _tail():
        pltpu.sync_copy(tail_hbm, in1_v.at[pl.ds(0, 16), pl.ds(0, _SROW)])
        pltpu.sync_copy(in1_v.at[pl.ds(0, 16), pl.ds(0, _SROW)],
                        sup_hbm.at[pl.ds(_NSUP - 16, 16), :])


def _gbody(feat_hbm, fv_hbm, sup_hbm, bias_hbm, out_hbm,
           idx0_v, idx1_v, idxq0_v, idxq1_v, emb0_v, emb1_v,
           bias0_v, bias1_v, fv0_v, fv1_v, out_v,
           sem_e0, sem_e1, sem_b0, sem_b1):
    nc = 2
    wid = lax.axis_index("s") * nc + lax.axis_index("c")
    iota = lax.iota(jnp.int32, _L)
    m1 = iota < (_F - _L)
    mlast = iota == (_L - 1)
    zero_v = jnp.zeros((_L,), jnp.float32)
    sems_e = (sem_e0, sem_e1)
    sems_b = (sem_b0, sem_b1)
    idxs = (idx0_v, idx1_v)
    idxqs = (idxq0_v, idxq1_v)
    embs = (emb0_v, emb1_v)
    biass = (bias0_v, bias1_v)
    fvs = (fv0_v, fv1_v)

    def stage(s, c):
        i0 = wid * (_NCHUNK * _IPC) + c * _IPC
        pltpu.sync_copy(feat_hbm.at[pl.ds(i0, _IPC)], idxs[s])
        pltpu.sync_copy(fv_hbm.at[pl.ds(i0, _IPC)], fvs[s])
        for j in range(_IPC // _L):
            idxqs[s][pl.ds(j * _L, _L)] = (
                lax.shift_right_logical(idxs[s][pl.ds(j * _L, _L)], 2))
        for j0 in range(0, _IPC, 128):
            n = min(128, _IPC - j0)
            pltpu.async_copy(
                sup_hbm.at[idxqs[s].at[pl.ds(j0, n)]],
                embs[s].at[pl.ds(j0, n), :], sems_e[s])
            pltpu.async_copy(
                bias_hbm.at[idxs[s].at[pl.ds(j0, n)]],
                biass[s].at[pl.ds(j0, n)], sems_b[s])

    def drain(s):
        for j0 in range(0, _IPC, 128):
            n = min(128, _IPC - j0)
            pltpu.make_async_copy(
                sup_hbm.at[pl.ds(0, n), :],
                embs[s].at[pl.ds(j0, n), :], sems_e[s]).wait()
            pltpu.make_async_copy(
                bias_hbm.at[pl.ds(0, n)],
                biass[s].at[pl.ds(j0, n)], sems_b[s]).wait()

    stage(0, 0)

    @pl.loop(0, _NCHUNK // 2)
    def _chunkpair(cc):
      for s in range(2):
        c = cc * 2 + s

        @pl.when(c + 1 < _NCHUNK)
        def _():
            stage(1 - s, c + 1)

        drain(s)
        idx_c = idxs[s]
        fv_c = fvs[s]
        emb_c = embs[s]
        bias_c = biass[s]

        @pl.loop(0, _CHUNK)
        def _row(b):
            r0 = b * _F
            ix0 = r0 + iota
            ix1 = ix0 + _L
            vv0 = plsc.load_gather(fv_c, [ix0])
            vv1 = plsc.load_gather(fv_c, [ix1], mask=m1)
            qo0 = (plsc.load_gather(idx_c, [ix0]) & 3) * _D
            qo1 = (plsc.load_gather(idx_c, [ix1], mask=m1) & 3) * _D
            acc0 = zero_v
            acc1 = zero_v
            sq0 = zero_v
            sq1 = zero_v
            rsplat = jnp.full((_L,), 0, jnp.int32) + r0
            for f in range(_F):
                sv = (vv0 if f < _L else vv1)[f % _L]
                co = (qo0 if f < _L else qo1)[f % _L]
                rvec = rsplat + f
                c0 = co + iota
                e0 = plsc.load_gather(emb_c, [rvec, c0])
                e1 = plsc.load_gather(emb_c, [rvec, c0 + _L])
                ev0 = e0 * sv
                ev1 = e1 * sv
                acc0 = acc0 + ev0
                acc1 = acc1 + ev1
                sq0 = sq0 + ev0 * ev0
                sq1 = sq1 + ev1 * ev1
            fm = acc0 * acc0 - sq0 + acc1 * acc1 - sq1
            bb0 = plsc.load_gather(bias_c, [ix0])
            bb1 = plsc.load_gather(bias_c, [ix1], mask=m1)
            bvec = bb0 * vv0 + jnp.where(m1, bb1 * vv1, 0.0)
            t = lax.reduce_sum(0.5 * fm + bvec, axes=(0,))
            tv = jnp.full((_L,), 0.0, jnp.float32) + t
            plsc.store_scatter(out_v, [jnp.full((_L,), 0, jnp.int32) + b],
                               tv, mask=mlast)

        pltpu.sync_copy(out_v, out_hbm.at[pl.ds(wid * _RPW + c * _CHUNK,
                                                _CHUNK)])


_MESH = plsc.VectorSubcoreMesh(core_axis_name="c", subcore_axis_name="s")
_PARAMS = pltpu.CompilerParams(
    needs_layout_passes=False, use_tc_tiling_on_sc=True)


@jax.jit
def _pointfm_sc(feat_flat, fv_flat, emb_t, tail16, bias_flat):
    sup = pl.kernel(
        _tbody,
        out_type=jax.ShapeDtypeStruct((_NSUP, _SROW), jnp.float32),
        mesh=_MESH,
        scratch_types=[
            pltpu.VMEM((_D, _W), jnp.float32),
            pltpu.VMEM((_D, _W), jnp.float32),
            pltpu.VMEM((_D, _W), jnp.float32),
            pltpu.VMEM((_D, _W), jnp.float32),
            pltpu.VMEM((_W // 4, _SROW), jnp.float32),
            pltpu.VMEM((_W // 4, _SROW), jnp.float32),
            pltpu.VMEM((_W // 4, _SROW), jnp.float32),
            pltpu.VMEM((_W // 4, _SROW), jnp.float32),
            pltpu.SemaphoreType.DMA,
            pltpu.SemaphoreType.DMA,
            pltpu.SemaphoreType.DMA,
            pltpu.SemaphoreType.DMA,
            pltpu.SemaphoreType.DMA,
            pltpu.SemaphoreType.DMA,
            pltpu.SemaphoreType.DMA,
            pltpu.SemaphoreType.DMA,
        ],
        compiler_params=_PARAMS,
    )(emb_t, tail16)
    return pl.kernel(
        _gbody,
        out_type=jax.ShapeDtypeStruct((_B,), jnp.float32),
        mesh=_MESH,
        scratch_types=[
            pltpu.VMEM((_IPC,), jnp.int32),
            pltpu.VMEM((_IPC,), jnp.int32),
            pltpu.VMEM((_IPC,), jnp.int32),
            pltpu.VMEM((_IPC,), jnp.int32),
            pltpu.VMEM((_IPC, _SROW), jnp.float32),
            pltpu.VMEM((_IPC, _SROW), jnp.float32),
            pltpu.VMEM((_IPC,), jnp.float32),
            pltpu.VMEM((_IPC,), jnp.float32),
            pltpu.VMEM((_IPC,), jnp.float32),
            pltpu.VMEM((_IPC,), jnp.float32),
            pltpu.VMEM((_CHUNK,), jnp.float32),
            pltpu.SemaphoreType.DMA,
            pltpu.SemaphoreType.DMA,
            pltpu.SemaphoreType.DMA,
            pltpu.SemaphoreType.DMA,
        ],
        compiler_params=_PARAMS,
    )(feat_flat, fv_flat, sup, bias_flat)


def kernel(features, feature_values, emb_table, bias_table, bias_):
    tail16 = emb_table[_V - 2 * _D:].reshape(16, _SROW)
    out = _pointfm_sc(features.reshape(-1), feature_values.reshape(-1),
                      emb_table.T, tail16, bias_table.reshape(-1))
    return out + bias_
